# Initial kernel scaffold; baseline (speedup 1.0000x reference)
import jax
import jax.numpy as jnp
from jax.experimental import pallas as pl


def kernel(input, weight, coordinates):
    raise NotImplementedError("write your pallas kernel here")


# speedup vs baseline: 3.4769x; 3.4769x over previous
import functools

import jax
import jax.numpy as jnp
from jax import lax
from jax.experimental import pallas as pl
from jax.experimental.pallas import tpu as pltpu
from jax.experimental.pallas import tpu_sc as plsc

_NTOKEN = 1000000
_EMSIZE = 128
_NUM_BASIS = 2
_NUM_CLUSTERS = 400
_HALF = _EMSIZE // _NUM_BASIS

_B = 4096 * 200
_NWORKERS = 32
_TOK_PER_W = _B // _NWORKERS
_CHUNK = 512
_NCHUNKS = _TOK_PER_W // _CHUNK
_GROWS = 2 * _CHUNK // 128

_mesh = plsc.VectorSubcoreMesh(core_axis_name="c", subcore_axis_name="s")


@functools.partial(
    pl.kernel,
    out_type=jax.ShapeDtypeStruct((2 * _B, _HALF), jnp.float32),
    mesh=_mesh,
    scratch_types=[
        pltpu.VMEM((_CHUNK,), jnp.int32),
        pltpu.VMEM((_GROWS, 128), jnp.int32),
        pltpu.VMEM((2 * _CHUNK,), jnp.int32),
        pltpu.VMEM((_GROWS, 128), jnp.int32),
        pltpu.VMEM((2 * _CHUNK, _HALF), jnp.float32),
        pltpu.SemaphoreType.DMA,
    ],
    compiler_params=pltpu.CompilerParams(use_tc_tiling_on_sc=False),
)
def _sc_embed(inp_hbm, coords_hbm, wtab_hbm, out_hbm, tbuf, fbuf, cbuf, ibuf,
              obuf, sem):
    wid = lax.axis_index("s") * 2 + lax.axis_index("c")
    tok0 = wid * _TOK_PER_W

    lane = lax.iota(jnp.int32, 16)
    rep_lo = lane >> 1
    rep_hi = rep_lo + 8
    basis = lane & 1

    _dn = lax.GatherDimensionNumbers(
        offset_dims=(), collapsed_slice_dims=(0,), start_index_map=(0,))

    def _lane_take(vec, idx):
        return lax.gather(vec, idx[:, None], _dn, slice_sizes=(1,),
                          mode=lax.GatherScatterMode.PROMISE_IN_BOUNDS)

    def chunk_body(ch, _):
        pltpu.sync_copy(inp_hbm.at[pl.ds(tok0 + ch * _CHUNK, _CHUNK)], tbuf)

        def fx_body(g, _):
            tv = tbuf[pl.ds(16 * g, 16)]
            lo = 2 * _lane_take(tv, rep_lo) + basis
            hi = 2 * _lane_take(tv, rep_hi) + basis
            fbuf[g >> 2, pl.ds((g & 3) * 32, 16)] = lo
            fbuf[g >> 2, pl.ds((g & 3) * 32 + 16, 16)] = hi
            return 0

        lax.fori_loop(0, _CHUNK // 16, fx_body, 0)

        cps = [
            pltpu.async_copy(coords_hbm.at[fbuf.at[j]],
                             cbuf.at[pl.ds(128 * j, 128)], sem)
            for j in range(_GROWS)
        ]
        for cp in cps:
            cp.wait()

        def ix_body(g, _):
            v = cbuf[pl.ds(16 * g, 16)]
            ibuf[g >> 3, pl.ds((g & 7) * 16, 16)] = v + basis * _NUM_CLUSTERS
            return 0

        lax.fori_loop(0, 2 * _CHUNK // 16, ix_body, 0)

        gps = [
            pltpu.async_copy(wtab_hbm.at[ibuf.at[j]],
                             obuf.at[pl.ds(128 * j, 128), :], sem)
            for j in range(_GROWS)
        ]
        for gp in gps:
            gp.wait()

        obase = 2 * (tok0 + ch * _CHUNK)
        pltpu.sync_copy(obuf, out_hbm.at[pl.ds(obase, 2 * _CHUNK), :])
        return 0

    lax.fori_loop(0, _NCHUNKS, chunk_body, 0)


def kernel(input, weight, coordinates):
    inp = input.reshape(-1).astype(jnp.int32)
    wtab = weight.reshape(_NUM_BASIS * _NUM_CLUSTERS, _HALF)
    coords_flat = coordinates.astype(jnp.int32).reshape(-1)
    out = _sc_embed(inp, coords_flat, wtab)
    return out.reshape(input.shape[0], input.shape[1], _EMSIZE)
